# Initial kernel scaffold; baseline (speedup 1.0000x reference)
import jax
import jax.numpy as jnp
from jax.experimental import pallas as pl


def kernel(delta, score):
    raise NotImplementedError("write your pallas kernel here")


# speedup vs baseline: 15.5307x; 15.5307x over previous
import functools

import numpy as np
import jax
import jax.numpy as jnp
from jax.experimental import pallas as pl
from jax.experimental.pallas import tpu as pltpu

_ANCHOR_SCALE = (8.0, 16.0, 32.0)
_ANCHOR_RATIO = (0.5, 1.0, 2.0)
_FEAT_STRIDE = 16
_RPN_MIN_SIZE = 16.0
_PRE_NMS_N = 6000
_POST_NMS_N = 300
_NMS_THRESH = 0.7
_IMG_W = 800.0
_IMG_H = 800.0
_MAP = 50
_N = _MAP * _MAP * 9
_ROWS = 176
_COLS = 128
_NP = _ROWS * _COLS


def _np_anchors():
    base_size = 16.0
    base = np.array([0.0, 0.0, base_size - 1.0, base_size - 1.0], dtype=np.float32)
    w = base[2] - base[0] + 1.0
    h = base[3] - base[1] + 1.0
    x_ctr = base[0] + 0.5 * (w - 1.0)
    y_ctr = base[1] + 0.5 * (h - 1.0)
    size = w * h
    anchors = []
    for r in _ANCHOR_RATIO:
        size_r = size / r
        ws = np.round(np.sqrt(size_r))
        hs = np.round(ws * r)
        for s in _ANCHOR_SCALE:
            wss = ws * s
            hss = hs * s
            anchors.append([x_ctr - 0.5 * (wss - 1.0), y_ctr - 0.5 * (hss - 1.0),
                            x_ctr + 0.5 * (wss - 1.0), y_ctr + 0.5 * (hss - 1.0)])
    anchors = np.array(anchors, dtype=np.float32)
    sx = np.arange(_MAP, dtype=np.float32) * _FEAT_STRIDE
    sy = np.arange(_MAP, dtype=np.float32) * _FEAT_STRIDE
    sxg, syg = np.meshgrid(sx, sy)
    shifts = np.stack([sxg.ravel(), syg.ravel(), sxg.ravel(), syg.ravel()], axis=1)
    all_anchors = (anchors[None, :, :] + shifts[:, None, :]).reshape(-1, 4)
    return all_anchors.astype(np.float32)


def _pad_grid(col, fill):
    out = np.full((_NP,), fill, dtype=np.float32)
    out[:_N] = col
    return out.reshape(_ROWS, _COLS)


@functools.lru_cache(maxsize=1)
def _anchor_consts():
    a = _np_anchors()
    widths = a[:, 2] - a[:, 0] + 1.0
    heights = a[:, 3] - a[:, 1] + 1.0
    ctr_x = a[:, 0] + 0.5 * widths
    ctr_y = a[:, 1] + 0.5 * heights
    return (_pad_grid(widths, 16.0), _pad_grid(heights, 16.0),
            _pad_grid(ctr_x, 0.0), _pad_grid(ctr_y, 0.0))


def _sortable(f):
    m = jax.lax.bitcast_convert_type(f, jnp.int32)
    neg = jnp.bitwise_xor(jnp.bitwise_not(m), jnp.int32(-2**31))
    return jnp.where(m >= 0, m, neg)


def _body(dx_ref, dy_ref, dw_ref, dh_ref, s_ref, wa_ref, ha_ref, cx_ref, cy_ref,
          ob_ref, os_ref, valid_ref):
    dx = dx_ref[...]
    dy = dy_ref[...]
    dw = dw_ref[...]
    dh = dh_ref[...]
    s_in = s_ref[...]
    wa = wa_ref[...]
    ha = ha_ref[...]
    cxa = cx_ref[...]
    cya = cy_ref[...]

    lin = (jax.lax.broadcasted_iota(jnp.int32, (_ROWS, _COLS), 0) * _COLS
           + jax.lax.broadcasted_iota(jnp.int32, (_ROWS, _COLS), 1))
    is_real = lin < _N

    pred_ctr_x = dx * wa + cxa
    pred_ctr_y = dy * ha + cya
    pred_w = jnp.exp(dw) * wa
    pred_h = jnp.exp(dh) * ha
    x1 = pred_ctr_x - 0.5 * pred_w
    y1 = pred_ctr_y - 0.5 * pred_h
    x2 = pred_ctr_x + 0.5 * pred_w
    y2 = pred_ctr_y + 0.5 * pred_h
    x1 = jnp.clip(x1, 0.0, _IMG_W - 1.0)
    y1 = jnp.clip(y1, 0.0, _IMG_H - 1.0)
    x2 = jnp.clip(x2, 0.0, _IMG_W - 1.0)
    y2 = jnp.clip(y2, 0.0, _IMG_H - 1.0)
    ws = x2 - x1 + 1.0
    hs = y2 - y1 + 1.0
    size_ok = (ws >= _RPN_MIN_SIZE) & (hs >= _RPN_MIN_SIZE)
    s_eff = jnp.where(size_ok, s_in, jnp.float32(-1e9))
    s_eff = jnp.where(is_real, s_eff, -jnp.inf)
    areas = (x2 - x1) * (y2 - y1)

    v = _sortable(s_eff + 0.0)
    K = jnp.int32(_PRE_NMS_N)

    def cnt_gt(t):
        return jnp.sum((v > t).astype(jnp.int32))

    int_min = jnp.int32(-2**31)
    l0 = jnp.where(cnt_gt(jnp.int32(-1)) >= K, jnp.int32(-1), int_min)

    def bis_step(t, l):
        cand = l + (jnp.int32(1) << (jnp.int32(30) - t))
        return jnp.where(cnt_gt(cand) >= K, cand, l)

    lfin = jax.lax.fori_loop(0, 31, bis_step, l0)
    thresh_v = lfin + jnp.int32(1)
    n_gt = cnt_gt(thresh_v)
    need = K - n_gt
    eq = v == thresh_v

    def cnt_eq_below(cb):
        return jnp.sum((eq & (lin < cb)).astype(jnp.int32))

    def idx_step(t, l):
        cand = l + (jnp.int32(1) << (jnp.int32(14) - t))
        return jnp.where(cnt_eq_below(cand) < need, cand, l)

    cfin = jax.lax.fori_loop(0, 15, idx_step, jnp.int32(0)) + jnp.int32(1)
    member = (v > thresh_v) | (eq & (lin < cfin))

    neg_inf = jnp.float32(-jnp.inf)
    big_i = jnp.int32(2**30)

    def select(valid):
        masked = jnp.where(valid, s_eff, neg_inf)
        m = jnp.max(masked)
        idx = jnp.min(jnp.where(masked == m, lin, big_i))
        return m, idx

    def emit_and_suppress(t, i, s_out, valid):

        onehot = lin == i
        x1i = jnp.sum(jnp.where(onehot, x1, 0.0))
        y1i = jnp.sum(jnp.where(onehot, y1, 0.0))
        x2i = jnp.sum(jnp.where(onehot, x2, 0.0))
        y2i = jnp.sum(jnp.where(onehot, y2, 0.0))
        ai = jnp.sum(jnp.where(onehot, areas, 0.0))
        ob_ref[t, 0] = x1i
        ob_ref[t, 1] = y1i
        ob_ref[t, 2] = x2i
        ob_ref[t, 3] = y2i
        os_ref[t] = s_out
        xx1 = jnp.maximum(x1i, x1)
        yy1 = jnp.maximum(y1i, y1)
        xx2 = jnp.minimum(x2i, x2)
        yy2 = jnp.minimum(y2i, y2)
        iw = jnp.maximum(xx2 - xx1, 0.0)
        ih = jnp.maximum(yy2 - yy1, 0.0)
        inter = iw * ih
        iou = inter / (ai + areas - inter + jnp.float32(1e-8))
        return valid & (iou <= _NMS_THRESH) & jnp.logical_not(onehot)

    m0, i0 = select(member)
    valid_ref[...] = emit_and_suppress(0, i0, m0, member).astype(jnp.float32)

    def nms_step(t, carry):
        valid = valid_ref[...] != 0.0
        m, idx = select(valid)
        empty = m == neg_inf
        i = jnp.where(empty, i0, idx)
        s_out = jnp.where(empty, m0, m)
        valid_ref[...] = emit_and_suppress(t, i, s_out, valid).astype(jnp.float32)
        return carry

    jax.lax.fori_loop(1, _POST_NMS_N, nms_step, jnp.int32(0))


def kernel(delta, score):
    wa, ha, cxa, cya = _anchor_consts()
    d = jnp.reshape(delta, (-1, 4))
    pad = jnp.zeros((_NP - _N,), jnp.float32)

    def grid(col):
        return jnp.reshape(jnp.concatenate([col, pad]), (_ROWS, _COLS))

    dx = grid(d[:, 0])
    dy = grid(d[:, 1])
    dw = grid(d[:, 2])
    dh = grid(d[:, 3])
    s = jnp.reshape(score[:, :, :, 9:], (-1,))
    s = jnp.reshape(jnp.concatenate([s, jnp.full((_NP - _N,), -jnp.inf)]),
                    (_ROWS, _COLS))

    out_boxes, out_scores = pl.pallas_call(
        _body,
        out_shape=(jax.ShapeDtypeStruct((_POST_NMS_N, 4), jnp.float32),
                   jax.ShapeDtypeStruct((_POST_NMS_N,), jnp.float32)),
        out_specs=(pl.BlockSpec(memory_space=pltpu.SMEM),
                   pl.BlockSpec(memory_space=pltpu.SMEM)),
        scratch_shapes=[pltpu.VMEM((_ROWS, _COLS), jnp.float32)],
    )(dx, dy, dw, dh, s,
      jnp.asarray(wa), jnp.asarray(ha), jnp.asarray(cxa), jnp.asarray(cya))
    return (out_boxes, out_scores)
